# baseline (device time: 38346 ns/iter reference)
import jax
import jax.numpy as jnp
from jax import lax
from jax.experimental import pallas as pl
from jax.experimental.pallas import tpu as pltpu

N_DEV = 4
SQ_SH = 256
SKV_USED = 1024
HQ_SH = 8
DH = 128
D_MODEL = 1024
QBLK = 64
SCALE = 0.08838834764831843
NEG = -1e9


def _body(x_ref, wq_ref, kext_ref, vext_ref, wo_ref, out_ref,
          ag_ref, part_ref, stg_ref, rcv_ref, ctx_ref, ctx2_ref,
          k32_ref, v32_ref, kbf_ref, vbf_ref,
          wq32_ref, wo32_ref, wqbf_ref, wobf_ref, bias_ref,
          ag_send, ag_recv, rs_send, rs_recv, kv_sem, w_sem):
    my = lax.axis_index("i")
    left = (my + N_DEV - 1) % N_DEV
    right = (my + 1) % N_DEV
    diag = (my + 2) % N_DEV

    base = my * HQ_SH
    kcp = [pltpu.make_async_copy(
        kext_ref.at[0, pl.ds(0, SKV_USED), base + h, :],
        k32_ref.at[h], kv_sem.at[0, h]) for h in range(HQ_SH)]
    vcp = [pltpu.make_async_copy(
        vext_ref.at[0, pl.ds(0, SKV_USED), base + h, :],
        v32_ref.at[h], kv_sem.at[1, h]) for h in range(HQ_SH)]
    wqcp = pltpu.make_async_copy(wq_ref, wq32_ref, w_sem.at[0])
    wocp = pltpu.make_async_copy(wo_ref, wo32_ref, w_sem.at[1])
    wqcp.start()
    wocp.start()
    for c in kcp:
        c.start()
    for c in vcp:
        c.start()

    barrier = pltpu.get_barrier_semaphore()
    for nbr in (left, right, diag):
        pl.semaphore_signal(barrier, inc=1, device_id=(nbr,),
                            device_id_type=pl.DeviceIdType.MESH)
    pl.semaphore_wait(barrier, 3)

    def mk(src, dst, ssem, rsem, dev):
        return pltpu.make_async_remote_copy(
            src_ref=src, dst_ref=dst, send_sem=ssem, recv_sem=rsem,
            device_id=(dev,), device_id_type=pl.DeviceIdType.MESH)

    ag_ref[0] = x_ref[:]
    agR = mk(ag_ref.at[0], ag_ref.at[1], ag_send.at[0], ag_recv.at[0], right)
    agL = mk(ag_ref.at[0], ag_ref.at[2], ag_send.at[1], ag_recv.at[1], left)
    agD = mk(ag_ref.at[0], ag_ref.at[3], ag_send.at[2], ag_recv.at[2], diag)
    agR.start()
    agL.start()

    wqcp.wait()
    wqbf_ref[:] = (wq32_ref[:] * SCALE).astype(jnp.bfloat16)
    for h in range(HQ_SH):
        kcp[h].wait()
        kbf_ref[h] = k32_ref[h].astype(jnp.bfloat16)
        vcp[h].wait()
        vbf_ref[h] = v32_ref[h].astype(jnp.bfloat16)
    wocp.wait()
    wobf_ref[:] = wo32_ref[:].astype(jnp.bfloat16)

    qb0 = lax.broadcasted_iota(jnp.int32, (SQ_SH, SKV_USED), 0) // QBLK
    kb0 = lax.broadcasted_iota(jnp.int32, (SQ_SH, SKV_USED), 1) // QBLK
    for i, rel in enumerate((0, 3, 1, 2)):
        origin = (my + rel) % N_DEV
        bias_ref[i] = jnp.where(kb0 <= qb0 + origin * (SQ_SH // QBLK),
                                0.0, NEG)

    def compute_chunk(xc, bias_slot, dst, f32_out):
        q = jnp.dot(xc, wqbf_ref[:], preferred_element_type=jnp.float32)
        neg = bias_ref[bias_slot]
        for h in range(HQ_SH):
            qh = q[:, h * DH:(h + 1) * DH].astype(jnp.bfloat16)
            s = lax.dot_general(qh, kbf_ref[h], (((1,), (1,)), ((), ())),
                                preferred_element_type=jnp.float32)
            w = jnp.exp(s + neg)
            denom = 1.0 / jnp.sum(w, axis=1, keepdims=True)
            ctx = jnp.dot(w.astype(jnp.bfloat16), vbf_ref[h],
                          preferred_element_type=jnp.float32)
            ctx_ref[:, h * DH:(h + 1) * DH] = (ctx * denom).astype(jnp.bfloat16)
        part = jnp.dot(ctx_ref[:], wobf_ref[:],
                       preferred_element_type=jnp.float32)
        dst[:] = part if f32_out else part.astype(jnp.bfloat16)

    agD.start()

    agR.wait_recv()
    agL.wait_recv()
    q2 = jnp.dot(ag_ref[1:3].reshape(2 * SQ_SH, D_MODEL), wqbf_ref[:],
                 preferred_element_type=jnp.float32)
    neg2 = bias_ref[1:3].reshape(2 * SQ_SH, SKV_USED)
    for h in range(HQ_SH):
        qh = q2[:, h * DH:(h + 1) * DH].astype(jnp.bfloat16)
        s = lax.dot_general(qh, kbf_ref[h], (((1,), (1,)), ((), ())),
                            preferred_element_type=jnp.float32)
        w = jnp.exp(s + neg2)
        denom = 1.0 / jnp.sum(w, axis=1, keepdims=True)
        ctx = jnp.dot(w.astype(jnp.bfloat16), vbf_ref[h],
                      preferred_element_type=jnp.float32)
        ctx2_ref[:, h * DH:(h + 1) * DH] = (ctx * denom).astype(jnp.bfloat16)
    p2 = jnp.dot(ctx2_ref[:], wobf_ref[:],
                 preferred_element_type=jnp.float32)
    stg_ref[1] = p2[:SQ_SH].astype(jnp.bfloat16)
    stg_ref[0] = p2[SQ_SH:].astype(jnp.bfloat16)
    rsL = mk(stg_ref.at[1], rcv_ref.at[1], rs_send.at[1], rs_recv.at[1], left)
    rsR = mk(stg_ref.at[0], rcv_ref.at[0], rs_send.at[0], rs_recv.at[0], right)
    rsL.start()
    rsR.start()

    agD.wait_recv()
    compute_chunk(ag_ref[3], 3, stg_ref.at[2], False)
    rsD = mk(stg_ref.at[2], rcv_ref.at[2], rs_send.at[2], rs_recv.at[2], diag)
    rsD.start()

    compute_chunk(ag_ref[0], 0, part_ref, True)

    rsR.wait_recv()
    rsL.wait_recv()
    rsD.wait_recv()
    out_ref[:] = ((part_ref[:] + rcv_ref[0].astype(jnp.float32))
                  + (rcv_ref[1].astype(jnp.float32)
                     + rcv_ref[2].astype(jnp.float32))).astype(jnp.bfloat16)

    agR.wait_send()
    agL.wait_send()
    agD.wait_send()
    rsR.wait_send()
    rsL.wait_send()
    rsD.wait_send()


def kernel(x, Wq, K_ext, V_ext, Wo):
    xb = x.reshape(SQ_SH, D_MODEL).astype(jnp.bfloat16)

    out = pl.pallas_call(
        _body,
        out_shape=jax.ShapeDtypeStruct((SQ_SH, D_MODEL), jnp.bfloat16),
        in_specs=[
            pl.BlockSpec(memory_space=pltpu.VMEM),
            pl.BlockSpec(memory_space=pl.ANY),
            pl.BlockSpec(memory_space=pl.ANY),
            pl.BlockSpec(memory_space=pl.ANY),
            pl.BlockSpec(memory_space=pl.ANY),
        ],
        out_specs=pl.BlockSpec(memory_space=pltpu.VMEM),
        scratch_shapes=[
            pltpu.VMEM((N_DEV, SQ_SH, D_MODEL), jnp.bfloat16),
            pltpu.VMEM((SQ_SH, D_MODEL), jnp.float32),
            pltpu.VMEM((N_DEV - 1, SQ_SH, D_MODEL), jnp.bfloat16),
            pltpu.VMEM((N_DEV - 1, SQ_SH, D_MODEL), jnp.bfloat16),
            pltpu.VMEM((SQ_SH, D_MODEL), jnp.bfloat16),
            pltpu.VMEM((2 * SQ_SH, D_MODEL), jnp.bfloat16),
            pltpu.VMEM((HQ_SH, SKV_USED, DH), jnp.float32),
            pltpu.VMEM((HQ_SH, SKV_USED, DH), jnp.float32),
            pltpu.VMEM((HQ_SH, SKV_USED, DH), jnp.bfloat16),
            pltpu.VMEM((HQ_SH, SKV_USED, DH), jnp.bfloat16),
            pltpu.VMEM((D_MODEL, D_MODEL), jnp.float32),
            pltpu.VMEM((D_MODEL, D_MODEL), jnp.float32),
            pltpu.VMEM((D_MODEL, D_MODEL), jnp.bfloat16),
            pltpu.VMEM((D_MODEL, D_MODEL), jnp.bfloat16),
            pltpu.VMEM((N_DEV, SQ_SH, SKV_USED), jnp.float32),
            pltpu.SemaphoreType.DMA((N_DEV - 1,)),
            pltpu.SemaphoreType.DMA((N_DEV - 1,)),
            pltpu.SemaphoreType.DMA((N_DEV - 1,)),
            pltpu.SemaphoreType.DMA((N_DEV - 1,)),
            pltpu.SemaphoreType.DMA((2, HQ_SH)),
            pltpu.SemaphoreType.DMA((2,)),
        ],
        compiler_params=pltpu.CompilerParams(
            collective_id=0, vmem_limit_bytes=64 * 1024 * 1024),
    )(xb, Wq, K_ext, V_ext, Wo)
    return out.reshape(1, SQ_SH, D_MODEL)


# device time: 36718 ns/iter; 1.0443x vs baseline; 1.0443x over previous
import jax
import jax.numpy as jnp
from jax import lax
from jax.experimental import pallas as pl
from jax.experimental.pallas import tpu as pltpu

N_DEV = 4
SQ_SH = 256
SKV_USED = 1024
HQ_SH = 8
DH = 128
D_MODEL = 1024
QBLK = 64
SCALE = 0.08838834764831843
NEG = -1e9


def _body(x_ref, wq_ref, kext_ref, vext_ref, wo_ref, out_ref,
          ag_ref, part_ref, stg_ref, rcv_ref, ctx_ref,
          k32_ref, v32_ref, kbf_ref, vbf_ref,
          wq32_ref, wo32_ref, wqbf_ref, wobf_ref, bias_ref,
          ag_send, ag_recv, rs_send, rs_recv, kv_sem, w_sem):
    my = lax.axis_index("i")
    left = (my + N_DEV - 1) % N_DEV
    right = (my + 1) % N_DEV
    diag = (my + 2) % N_DEV

    base = my * HQ_SH
    kcp = [pltpu.make_async_copy(
        kext_ref.at[0, pl.ds(0, SKV_USED), base + h, :],
        k32_ref.at[h], kv_sem.at[0, h]) for h in range(HQ_SH)]
    vcp = [pltpu.make_async_copy(
        vext_ref.at[0, pl.ds(0, SKV_USED), base + h, :],
        v32_ref.at[h], kv_sem.at[1, h]) for h in range(HQ_SH)]
    wqcp = pltpu.make_async_copy(wq_ref, wq32_ref, w_sem.at[0])
    wocp = pltpu.make_async_copy(wo_ref, wo32_ref, w_sem.at[1])
    wqcp.start()
    wocp.start()
    for c in kcp:
        c.start()
    for c in vcp:
        c.start()

    barrier = pltpu.get_barrier_semaphore()
    for nbr in (left, right, diag):
        pl.semaphore_signal(barrier, inc=1, device_id=(nbr,),
                            device_id_type=pl.DeviceIdType.MESH)
    pl.semaphore_wait(barrier, 3)

    def mk(src, dst, ssem, rsem, dev):
        return pltpu.make_async_remote_copy(
            src_ref=src, dst_ref=dst, send_sem=ssem, recv_sem=rsem,
            device_id=(dev,), device_id_type=pl.DeviceIdType.MESH)

    ag_ref[0] = x_ref[:]
    agR = mk(ag_ref.at[0], ag_ref.at[3], ag_send.at[0], ag_recv.at[0], right)
    agL = mk(ag_ref.at[0], ag_ref.at[1], ag_send.at[1], ag_recv.at[1], left)
    agD = mk(ag_ref.at[0], ag_ref.at[2], ag_send.at[2], ag_recv.at[2], diag)
    agR.start()
    agL.start()

    wqcp.wait()
    wqbf_ref[:] = (wq32_ref[:] * SCALE).astype(jnp.bfloat16)
    for h in range(HQ_SH):
        kcp[h].wait()
        kbf_ref[h] = k32_ref[h].astype(jnp.bfloat16)
        vcp[h].wait()
        vbf_ref[h] = v32_ref[h].astype(jnp.bfloat16)
    wocp.wait()
    wobf_ref[:] = wo32_ref[:].astype(jnp.bfloat16)

    qb0 = lax.broadcasted_iota(jnp.int32, (SQ_SH, SKV_USED), 0) // QBLK
    kb0 = lax.broadcasted_iota(jnp.int32, (SQ_SH, SKV_USED), 1) // QBLK
    for i, rel in enumerate((0, 3, 1, 2)):
        origin = (my + rel) % N_DEV
        bias_ref[i] = jnp.where(kb0 <= qb0 + origin * (SQ_SH // QBLK),
                                0.0, NEG)

    def compute_chunk(xc, bias_slot, dst, f32_out):
        q = jnp.dot(xc, wqbf_ref[:], preferred_element_type=jnp.float32)
        neg = bias_ref[bias_slot]
        for h in range(HQ_SH):
            qh = q[:, h * DH:(h + 1) * DH].astype(jnp.bfloat16)
            s = lax.dot_general(qh, kbf_ref[h], (((1,), (1,)), ((), ())),
                                preferred_element_type=jnp.float32)
            w = jnp.exp(s + neg)
            denom = 1.0 / jnp.sum(w, axis=1, keepdims=True)
            ctx = jnp.dot(w.astype(jnp.bfloat16), vbf_ref[h],
                          preferred_element_type=jnp.float32)
            ctx_ref[:, h * DH:(h + 1) * DH] = (ctx * denom).astype(jnp.bfloat16)
        part = jnp.dot(ctx_ref[:], wobf_ref[:],
                       preferred_element_type=jnp.float32)
        dst[:] = part if f32_out else part.astype(jnp.bfloat16)

    agD.start()

    agR.wait_recv()
    compute_chunk(ag_ref[3], 1, stg_ref.at[1], False)
    rsL = mk(stg_ref.at[1], rcv_ref.at[1], rs_send.at[1], rs_recv.at[1], left)
    rsL.start()

    agL.wait_recv()
    compute_chunk(ag_ref[1], 2, stg_ref.at[0], False)
    rsR = mk(stg_ref.at[0], rcv_ref.at[0], rs_send.at[0], rs_recv.at[0], right)
    rsR.start()

    agD.wait_recv()
    compute_chunk(ag_ref[2], 3, stg_ref.at[2], False)
    rsD = mk(stg_ref.at[2], rcv_ref.at[2], rs_send.at[2], rs_recv.at[2], diag)
    rsD.start()

    compute_chunk(ag_ref[0], 0, part_ref, True)

    rsR.wait_recv()
    rsL.wait_recv()
    rsD.wait_recv()
    out_ref[:] = ((part_ref[:] + rcv_ref[0].astype(jnp.float32))
                  + (rcv_ref[1].astype(jnp.float32)
                     + rcv_ref[2].astype(jnp.float32))).astype(jnp.bfloat16)

    agR.wait_send()
    agL.wait_send()
    agD.wait_send()
    rsR.wait_send()
    rsL.wait_send()
    rsD.wait_send()


def kernel(x, Wq, K_ext, V_ext, Wo):
    xb = x.reshape(SQ_SH, D_MODEL).astype(jnp.bfloat16)

    out = pl.pallas_call(
        _body,
        out_shape=jax.ShapeDtypeStruct((SQ_SH, D_MODEL), jnp.bfloat16),
        in_specs=[
            pl.BlockSpec(memory_space=pltpu.VMEM),
            pl.BlockSpec(memory_space=pl.ANY),
            pl.BlockSpec(memory_space=pl.ANY),
            pl.BlockSpec(memory_space=pl.ANY),
            pl.BlockSpec(memory_space=pl.ANY),
        ],
        out_specs=pl.BlockSpec(memory_space=pltpu.VMEM),
        scratch_shapes=[
            pltpu.VMEM((N_DEV, SQ_SH, D_MODEL), jnp.bfloat16),
            pltpu.VMEM((SQ_SH, D_MODEL), jnp.float32),
            pltpu.VMEM((N_DEV - 1, SQ_SH, D_MODEL), jnp.bfloat16),
            pltpu.VMEM((N_DEV - 1, SQ_SH, D_MODEL), jnp.bfloat16),
            pltpu.VMEM((SQ_SH, D_MODEL), jnp.bfloat16),
            pltpu.VMEM((HQ_SH, SKV_USED, DH), jnp.float32),
            pltpu.VMEM((HQ_SH, SKV_USED, DH), jnp.float32),
            pltpu.VMEM((HQ_SH, SKV_USED, DH), jnp.bfloat16),
            pltpu.VMEM((HQ_SH, SKV_USED, DH), jnp.bfloat16),
            pltpu.VMEM((D_MODEL, D_MODEL), jnp.float32),
            pltpu.VMEM((D_MODEL, D_MODEL), jnp.float32),
            pltpu.VMEM((D_MODEL, D_MODEL), jnp.bfloat16),
            pltpu.VMEM((D_MODEL, D_MODEL), jnp.bfloat16),
            pltpu.VMEM((N_DEV, SQ_SH, SKV_USED), jnp.float32),
            pltpu.SemaphoreType.DMA((N_DEV - 1,)),
            pltpu.SemaphoreType.DMA((N_DEV - 1,)),
            pltpu.SemaphoreType.DMA((N_DEV - 1,)),
            pltpu.SemaphoreType.DMA((N_DEV - 1,)),
            pltpu.SemaphoreType.DMA((2, HQ_SH)),
            pltpu.SemaphoreType.DMA((2,)),
        ],
        compiler_params=pltpu.CompilerParams(
            collective_id=0, vmem_limit_bytes=64 * 1024 * 1024),
    )(xb, Wq, K_ext, V_ext, Wo)
    return out.reshape(1, SQ_SH, D_MODEL)
